# baseline (speedup 1.0000x reference)
import functools

import jax
import jax.numpy as jnp
from jax import lax
from jax.experimental import pallas as pl
from jax.experimental.pallas import tpu as pltpu
from jax.experimental.pallas import tpu_sc as plsc

VOCAB = 100000
EMBED_DIM = 128
BATCH = 4096
SEQ_LEN = 200

NC = 2
NS = 16
NW = NC * NS

TOTAL = BATCH * SEQ_LEN
PER_W = TOTAL // NW
CHUNK = 128
NSTEPS = PER_W // CHUNK


NSLOT = 4
PAIR = 2


def _body(x_hbm, table_hbm, out_hbm, idx_v, rows_v, sg, sw):
    wid = lax.axis_index("s") * NC + lax.axis_index("c")
    base = wid * PER_W
    pltpu.sync_copy(x_hbm.at[wid], idx_v)

    def gather(j, s):
        pltpu.async_copy(
            table_hbm.at[idx_v.at[j]],
            rows_v.at[pl.ds(s * CHUNK, CHUNK)], sg[s])

    def wait_gather(j, s):
        pltpu.make_async_copy(
            table_hbm.at[idx_v.at[j]],
            rows_v.at[pl.ds(s * CHUNK, CHUNK)], sg[s]).wait()

    def write_pair(p, h):
        return pltpu.async_copy(
            rows_v.at[pl.ds(h * PAIR * CHUNK, PAIR * CHUNK)],
            out_hbm.at[pl.ds(base + p * PAIR * CHUNK, PAIR * CHUNK)], sw[h])

    def do_pair(p, h):
        wait_gather(PAIR * p, PAIR * h)
        wait_gather(PAIR * p + 1, PAIR * h + 1)
        write_pair(p, h).wait()

    NPAIR = NSTEPS // PAIR
    for s in range(NSLOT):
        gather(s, s)

    def step(g):
        for h in range(2):
            p = g + h
            do_pair(p, h)
            gather(PAIR * p + NSLOT, PAIR * h)
            gather(PAIR * p + NSLOT + 1, PAIR * h + 1)

    pl.loop(0, NPAIR - 2, step=2)(step)

    do_pair(NPAIR - 2, 0)
    do_pair(NPAIR - 1, 1)


@jax.jit
def kernel(x, table):
    x3 = x.reshape(NW, NSTEPS, CHUNK).astype(jnp.int32)
    run = functools.partial(
        pl.kernel,
        out_type=jax.ShapeDtypeStruct((TOTAL, EMBED_DIM), jnp.float32),
        mesh=plsc.VectorSubcoreMesh(core_axis_name="c", subcore_axis_name="s"),
        scratch_types=[
            pltpu.VMEM((NSTEPS, CHUNK), jnp.int32),
            pltpu.VMEM((NSLOT * CHUNK, EMBED_DIM), jnp.float32),
            [pltpu.SemaphoreType.DMA] * NSLOT,
            [pltpu.SemaphoreType.DMA] * 2,
        ],
    )(_body)
    out = run(x3, table)
    return out.reshape(BATCH, SEQ_LEN, EMBED_DIM)
